# baseline (speedup 1.0000x reference)
import functools

import jax
import jax.numpy as jnp
from jax import lax
from jax.experimental import pallas as pl
from jax.experimental.pallas import tpu as pltpu
from jax.experimental.pallas import tpu_sc as plsc

_N = 10000
_F = 128
_E = 320000
_NC = 2
_NS = 16
_CHUNK = 64
_EPAD = 327680
_R = 10240
_RPS = _R // _NS
_NBUF = 4
_STG = 40
_T0 = 320


def _sc_segment_sum_body(x_hbm, src_hbm, dst_hbm, out_hbm,
                         srcidx_v, dstidx_v, rows_v, acc_sh, *gsems):
    cid = lax.axis_index("c")
    sid = lax.axis_index("s")

    with jax.named_scope("sc_zero_acc"):
        zeros16 = jnp.zeros((16,), jnp.float32)

        @pl.loop(0, _CHUNK)
        def _(i):
            @pl.loop(0, _F // 16)
            def _(j):
                rows_v[0, i, pl.ds(j * 16, 16)] = zeros16

        @pl.when(cid == 0)
        def _():
            @pl.loop(0, _RPS // _CHUNK)
            def _(k):
                pltpu.sync_copy(
                    rows_v.at[0],
                    acc_sh.at[pl.ds(sid * _RPS + k * _CHUNK, _CHUNK)])

        plsc.subcore_barrier()

    def _edge_stage(chunk_base):
        pltpu.sync_copy(src_hbm.at[pl.ds(chunk_base, _STG)], srcidx_v)
        pltpu.sync_copy(dst_hbm.at[pl.ds(chunk_base, _STG)], dstidx_v)

        for b in range(_NBUF):
            pltpu.async_copy(x_hbm.at[srcidx_v.at[b]], rows_v.at[b],
                             gsems[b])

        @pl.loop(0, _STG // _NBUF - 1)
        def _(u):
            for b in range(_NBUF):
                c = u * _NBUF + b
                pltpu.make_async_copy(x_hbm.at[srcidx_v.at[c]],
                                      rows_v.at[b], gsems[b]).wait()
                pltpu.sync_copy(rows_v.at[b], acc_sh.at[dstidx_v.at[c]],
                                add=True)
                pltpu.async_copy(x_hbm.at[srcidx_v.at[c + _NBUF]],
                                 rows_v.at[b], gsems[b])

        for b in range(_NBUF):
            c = _STG - _NBUF + b
            pltpu.make_async_copy(x_hbm.at[srcidx_v.at[c]], rows_v.at[b],
                                  gsems[b]).wait()
            pltpu.sync_copy(rows_v.at[b], acc_sh.at[dstidx_v.at[c]],
                            add=True)

    with jax.named_scope("sc_edge_loop"):
        @pl.when(cid == 0)
        def _():
            for h in range(_T0 // _STG):
                _edge_stage(sid * _T0 + h * _STG)

        plsc.subcore_barrier()

    with jax.named_scope("sc_writeout"):
        @pl.when(cid == 0)
        def _():
            @pl.loop(0, _RPS // _CHUNK)
            def _(k):
                off = sid * _RPS + k * _CHUNK
                pltpu.sync_copy(acc_sh.at[pl.ds(off, _CHUNK)],
                                out_hbm.at[pl.ds(off, _CHUNK)])


@functools.partial(
    pl.kernel,
    out_type=jax.ShapeDtypeStruct((_R, _F), jnp.float32),
    mesh=plsc.VectorSubcoreMesh(core_axis_name="c", subcore_axis_name="s"),
    scratch_types=[
        pltpu.VMEM((_STG, _CHUNK), jnp.int32),
        pltpu.VMEM((_STG, _CHUNK), jnp.int32),
        pltpu.VMEM((_NBUF, _CHUNK, _F), jnp.float32),
        pltpu.VMEM_SHARED((_R, _F), jnp.float32),
    ] + [pltpu.SemaphoreType.DMA] * _NBUF,
)
def _sc_segment_sum(x_hbm, src_hbm, dst_hbm, out_hbm,
                    srcidx_v, dstidx_v, rows_v, acc_sh, *gsems):
    _sc_segment_sum_body(x_hbm, src_hbm, dst_hbm, out_hbm,
                         srcidx_v, dstidx_v, rows_v, acc_sh, *gsems)


def _quat(w):
    r, i, j, k = jnp.split(w, 4, axis=1)
    r2 = jnp.concatenate([r, -i, -j, -k], axis=0)
    i2 = jnp.concatenate([i, r, -k, j], axis=0)
    j2 = jnp.concatenate([j, k, r, -i], axis=0)
    k2 = jnp.concatenate([k, -j, i, r], axis=0)
    return jnp.concatenate([r2, i2, j2, k2], axis=1)


def _tc_dense_body(part_ref, w1_ref, w2_ref, g_ref, b_ref, out_ref):
    x = part_ref[:_N, :]
    h1 = _quat(w1_ref[...])
    o1 = jnp.dot(x, h1, preferred_element_type=jnp.float32)
    mean = jnp.mean(o1, axis=0, keepdims=True)
    var = jnp.mean((o1 - mean) ** 2, axis=0, keepdims=True)
    o1 = (o1 - mean) * lax.rsqrt(var + 1e-5) * g_ref[...] + b_ref[...]
    o1 = jnp.tanh(o1)
    h2 = _quat(w2_ref[...])
    out_ref[...] = jnp.dot(o1, h2, preferred_element_type=jnp.float32)


def kernel(input, edge_index, weight1, weight2, bn_gamma, bn_beta):
    src = edge_index[0]
    dst = edge_index[1]
    pad = _EPAD - _E
    src_p = jnp.concatenate([src, jnp.zeros((pad,), jnp.int32)])
    dst_p = jnp.concatenate([dst, jnp.full((pad,), _N, jnp.int32)])
    src_p = src_p.reshape(_EPAD // _CHUNK, _CHUNK)
    dst_p = dst_p.reshape(_EPAD // _CHUNK, _CHUNK)

    partial = _sc_segment_sum(input, src_p, dst_p)

    out = pl.pallas_call(
        _tc_dense_body,
        out_shape=jax.ShapeDtypeStruct((_N, _F), jnp.float32),
    )(partial, weight1, weight2,
      bn_gamma.reshape(1, _F), bn_beta.reshape(1, _F))
    return out


# speedup vs baseline: 4.7444x; 4.7444x over previous
import functools

import jax
import jax.numpy as jnp
from jax import lax
from jax.experimental import pallas as pl
from jax.experimental.pallas import tpu as pltpu
from jax.experimental.pallas import tpu_sc as plsc

_N = 10000
_F = 128
_E = 320000
_NC = 2
_NS = 16
_CHUNK = 64
_EPAD = 327680
_R = 10240
_RPS = _R // _NS
_NBUF = 4
_STG = 40
_T0 = 160


def _sc_segment_sum_body(x_hbm, src_hbm, dst_hbm, out_hbm,
                         srcidx_v, dstidx_v, rows_v, acc_sh, *gsems):
    cid = lax.axis_index("c")
    sid = lax.axis_index("s")

    with jax.named_scope("sc_zero_acc"):
        zeros16 = jnp.zeros((16,), jnp.float32)

        @pl.loop(0, _CHUNK)
        def _(i):
            @pl.loop(0, _F // 16)
            def _(j):
                rows_v[0, i, pl.ds(j * 16, 16)] = zeros16

        @pl.loop(0, _RPS // _CHUNK)
        def _(k):
            pltpu.sync_copy(
                rows_v.at[0],
                acc_sh.at[pl.ds(sid * _RPS + k * _CHUNK, _CHUNK)])

        plsc.subcore_barrier()

    def _edge_stage(chunk_base):
        pltpu.sync_copy(src_hbm.at[pl.ds(chunk_base, _STG)], srcidx_v)
        pltpu.sync_copy(dst_hbm.at[pl.ds(chunk_base, _STG)], dstidx_v)

        for b in range(_NBUF):
            pltpu.async_copy(x_hbm.at[srcidx_v.at[b]], rows_v.at[b],
                             gsems[b])

        @pl.loop(0, _STG // _NBUF - 1)
        def _(u):
            for b in range(_NBUF):
                c = u * _NBUF + b
                pltpu.make_async_copy(x_hbm.at[srcidx_v.at[c]],
                                      rows_v.at[b], gsems[b]).wait()
                pltpu.sync_copy(rows_v.at[b], acc_sh.at[dstidx_v.at[c]],
                                add=True)
                pltpu.async_copy(x_hbm.at[srcidx_v.at[c + _NBUF]],
                                 rows_v.at[b], gsems[b])

        for b in range(_NBUF):
            c = _STG - _NBUF + b
            pltpu.make_async_copy(x_hbm.at[srcidx_v.at[c]], rows_v.at[b],
                                  gsems[b]).wait()
            pltpu.sync_copy(rows_v.at[b], acc_sh.at[dstidx_v.at[c]],
                            add=True)

    with jax.named_scope("sc_edge_loop"):
        wid = cid * _NS + sid
        for h in range(_T0 // _STG):
            _edge_stage(wid * _T0 + h * _STG)

        plsc.subcore_barrier()

    with jax.named_scope("sc_writeout"):
        @pl.loop(0, _RPS // _CHUNK)
        def _(k):
            off = sid * _RPS + k * _CHUNK
            pltpu.sync_copy(acc_sh.at[pl.ds(off, _CHUNK)],
                            out_hbm.at[cid].at[pl.ds(off, _CHUNK)])


@functools.partial(
    pl.kernel,
    out_type=jax.ShapeDtypeStruct((_NC, _R, _F), jnp.float32),
    mesh=plsc.VectorSubcoreMesh(core_axis_name="c", subcore_axis_name="s"),
    scratch_types=[
        pltpu.VMEM((_STG, _CHUNK), jnp.int32),
        pltpu.VMEM((_STG, _CHUNK), jnp.int32),
        pltpu.VMEM((_NBUF, _CHUNK, _F), jnp.float32),
        pltpu.VMEM_SHARED((_R, _F), jnp.float32),
    ] + [pltpu.SemaphoreType.DMA] * _NBUF,
)
def _sc_segment_sum(x_hbm, src_hbm, dst_hbm, out_hbm,
                    srcidx_v, dstidx_v, rows_v, acc_sh, *gsems):
    _sc_segment_sum_body(x_hbm, src_hbm, dst_hbm, out_hbm,
                         srcidx_v, dstidx_v, rows_v, acc_sh, *gsems)


def _quat(w):
    r, i, j, k = jnp.split(w, 4, axis=1)
    r2 = jnp.concatenate([r, -i, -j, -k], axis=0)
    i2 = jnp.concatenate([i, r, -k, j], axis=0)
    j2 = jnp.concatenate([j, k, r, -i], axis=0)
    k2 = jnp.concatenate([k, -j, i, r], axis=0)
    return jnp.concatenate([r2, i2, j2, k2], axis=1)


def _tc_dense_body(part_ref, w1_ref, w2_ref, g_ref, b_ref, out_ref):
    x = part_ref[0, :_N, :] + part_ref[1, :_N, :]
    h1 = _quat(w1_ref[...])
    o1 = jnp.dot(x, h1, preferred_element_type=jnp.float32)
    mean = jnp.mean(o1, axis=0, keepdims=True)
    var = jnp.mean((o1 - mean) ** 2, axis=0, keepdims=True)
    o1 = (o1 - mean) * lax.rsqrt(var + 1e-5) * g_ref[...] + b_ref[...]
    o1 = jnp.tanh(o1)
    h2 = _quat(w2_ref[...])
    out_ref[...] = jnp.dot(o1, h2, preferred_element_type=jnp.float32)


def kernel(input, edge_index, weight1, weight2, bn_gamma, bn_beta):
    src = edge_index[0]
    dst = edge_index[1]
    pad = _EPAD - _E
    it = lax.iota(jnp.int32, pad)
    src_p = jnp.concatenate([src, it % _N])
    dst_p = jnp.concatenate([dst, _N + (it % (_R - _N))])
    src_p = src_p.reshape(_EPAD // _CHUNK, _CHUNK)
    dst_p = dst_p.reshape(_EPAD // _CHUNK, _CHUNK)

    partial = _sc_segment_sum(input, src_p, dst_p)

    out = pl.pallas_call(
        _tc_dense_body,
        out_shape=jax.ShapeDtypeStruct((_N, _F), jnp.float32),
    )(partial, weight1, weight2,
      bn_gamma.reshape(1, _F), bn_beta.reshape(1, _F))
    return out
